# baseline (speedup 1.0000x reference)
import functools

import jax
import jax.numpy as jnp
from jax.experimental import pallas as pl
from jax.experimental.pallas import tpu as pltpu

R = 10
_TB = 512


def _matvec_kernel(x_ref, w_ref, o_ref):
    o_ref[...] = jnp.dot(x_ref[...], w_ref[...],
                         preferred_element_type=jnp.float32)


def _pool_kernel(len_ref, s_ref, o_ref):
    B, T = s_ref.shape
    t = jax.lax.broadcasted_iota(jnp.int32, (1, T), 1)
    for b in range(B):
        L = len_ref[b]
        s = s_ref[b:b + 1, :]
        acc = jnp.float32(0.0)
        for r in range(R):
            start = (r * L) // R
            end = ((r + 1) * L + R - 1) // R
            mask = (t >= start) & (t < end)
            acc = acc + jnp.max(jnp.where(mask, s, -jnp.inf))
            acc = acc + jnp.min(jnp.where(mask, s, jnp.inf))
        o_ref[b:b + 1, :] = jnp.full((1, 128), jax.nn.sigmoid(acc),
                                     dtype=jnp.float32)


@jax.jit
def kernel(x, lengths, W):
    B, T, D = x.shape
    xf = x.reshape(B * T, D)
    scores = pl.pallas_call(
        _matvec_kernel,
        grid=(B * T // _TB,),
        in_specs=[
            pl.BlockSpec((_TB, D), lambda i: (i, 0)),
            pl.BlockSpec((D, 1), lambda i: (0, 0)),
        ],
        out_specs=pl.BlockSpec((_TB, 1), lambda i: (i, 0)),
        out_shape=jax.ShapeDtypeStruct((B * T, 1), jnp.float32),
    )(xf, W)
    scores = scores.reshape(B, T)

    pooled = pl.pallas_call(
        _pool_kernel,
        in_specs=[
            pl.BlockSpec(memory_space=pltpu.SMEM),
            pl.BlockSpec((B, T), lambda: (0, 0)),
        ],
        out_specs=pl.BlockSpec((B, 128), lambda: (0, 0)),
        out_shape=jax.ShapeDtypeStruct((B, 128), jnp.float32),
    )(lengths, scores)
    return pooled[:, 0]


# speedup vs baseline: 1.1548x; 1.1548x over previous
import functools

import jax
import jax.numpy as jnp
from jax.experimental import pallas as pl
from jax.experimental.pallas import tpu as pltpu

R = 10
_TB = 2048


def _matvec_kernel(x_ref, w_ref, o_ref):
    o_ref[...] = jnp.dot(x_ref[...], w_ref[...],
                         preferred_element_type=jnp.float32)


def _pool_kernel(len_ref, s_ref, o_ref):
    B, T = s_ref.shape
    t = jax.lax.broadcasted_iota(jnp.int32, (1, T), 1)
    for b in range(B):
        L = len_ref[b]
        s = s_ref[b:b + 1, :]
        acc = jnp.float32(0.0)
        for r in range(R):
            start = (r * L) // R
            end = ((r + 1) * L + R - 1) // R
            mask = (t >= start) & (t < end)
            acc = acc + jnp.max(jnp.where(mask, s, -jnp.inf))
            acc = acc + jnp.min(jnp.where(mask, s, jnp.inf))
        o_ref[b:b + 1, :] = jnp.full((1, 128), jax.nn.sigmoid(acc),
                                     dtype=jnp.float32)


@jax.jit
def kernel(x, lengths, W):
    B, T, D = x.shape
    xf = x.reshape(B * T, D)
    scores = pl.pallas_call(
        _matvec_kernel,
        grid=(B * T // _TB,),
        in_specs=[
            pl.BlockSpec((_TB, D), lambda i: (i, 0)),
            pl.BlockSpec((D, 1), lambda i: (0, 0)),
        ],
        out_specs=pl.BlockSpec((_TB, 1), lambda i: (i, 0)),
        out_shape=jax.ShapeDtypeStruct((B * T, 1), jnp.float32),
    )(xf, W)
    scores = scores.reshape(B, T)

    pooled = pl.pallas_call(
        _pool_kernel,
        in_specs=[
            pl.BlockSpec(memory_space=pltpu.SMEM),
            pl.BlockSpec((B, T), lambda: (0, 0)),
        ],
        out_specs=pl.BlockSpec((B, 128), lambda: (0, 0)),
        out_shape=jax.ShapeDtypeStruct((B, 128), jnp.float32),
    )(lengths, scores)
    return pooled[:, 0]


# speedup vs baseline: 1.3368x; 1.1576x over previous
import jax
import jax.numpy as jnp
from jax.experimental import pallas as pl
from jax.experimental.pallas import tpu as pltpu

R = 10
_TB = 2048


def _matvec_kernel(x_ref, w_ref, o_ref):
    s = jnp.dot(x_ref[...], w_ref[...], preferred_element_type=jnp.float32)
    o_ref[...] = s.T[None]


def _pool_kernel(len_ref, s_ref, o_ref):
    BC, _, TB = s_ref.shape
    B = len_ref.shape[0]
    C = BC // B
    t = (jax.lax.broadcasted_iota(jnp.int32, (C, TB), 0) * TB
         + jax.lax.broadcasted_iota(jnp.int32, (C, TB), 1))
    for b in range(B):
        L = len_ref[b]
        s = s_ref[b * C:(b + 1) * C, 0, :]
        acc = jnp.float32(0.0)
        for r in range(R):
            start = (r * L) // R
            end = ((r + 1) * L + R - 1) // R
            mask = (t >= start) & (t < end)
            acc = acc + jnp.max(jnp.where(mask, s, -jnp.inf))
            acc = acc + jnp.min(jnp.where(mask, s, jnp.inf))
        o_ref[b:b + 1, :] = jnp.full((1, 128), jax.nn.sigmoid(acc),
                                     dtype=jnp.float32)


@jax.jit
def kernel(x, lengths, W):
    B, T, D = x.shape
    nt = T // _TB
    xf = x.reshape(B * T, D)
    scores = pl.pallas_call(
        _matvec_kernel,
        grid=(B * T // _TB,),
        in_specs=[
            pl.BlockSpec((_TB, D), lambda i: (i, 0)),
            pl.BlockSpec((D, 1), lambda i: (0, 0)),
        ],
        out_specs=pl.BlockSpec((1, 1, _TB), lambda i: (i, 0, 0)),
        out_shape=jax.ShapeDtypeStruct((B * nt, 1, _TB), jnp.float32),
    )(xf, W)

    pooled = pl.pallas_call(
        _pool_kernel,
        in_specs=[
            pl.BlockSpec(memory_space=pltpu.SMEM),
            pl.BlockSpec((B * nt, 1, _TB), lambda: (0, 0, 0)),
        ],
        out_specs=pl.BlockSpec((B, 128), lambda: (0, 0)),
        out_shape=jax.ShapeDtypeStruct((B, 128), jnp.float32),
    )(lengths, scores)
    return pooled[:, 0]


# speedup vs baseline: 1.3972x; 1.0452x over previous
import jax
import jax.numpy as jnp
from jax.experimental import pallas as pl
from jax.experimental.pallas import tpu as pltpu

R = 10
_TB = 2048


def _fused_kernel(len_ref, x_ref, w_ref, o_ref, smax_ref, smin_ref):
    i = pl.program_id(0)
    nchunks = pl.num_programs(0)
    TB = x_ref.shape[0]

    s = jnp.dot(x_ref[...], w_ref[...],
                preferred_element_type=jnp.float32).T

    cpb = nchunks // len_ref.shape[0]
    b = i // cpb
    c = i % cpb
    off = c * TB
    L = len_ref[b]

    t = off + jax.lax.broadcasted_iota(jnp.int32, (1, TB), 1)

    @pl.when(c == 0)
    def _init():
        for r in range(R):
            smax_ref[r] = jnp.float32(-jnp.inf)
            smin_ref[r] = jnp.float32(jnp.inf)

    for r in range(R):
        start = (r * L) // R
        end = ((r + 1) * L + R - 1) // R
        mask = (t >= start) & (t < end)
        cmax = jnp.max(jnp.where(mask, s, -jnp.inf))
        cmin = jnp.min(jnp.where(mask, s, jnp.inf))
        smax_ref[r] = jnp.maximum(smax_ref[r], cmax)
        smin_ref[r] = jnp.minimum(smin_ref[r], cmin)

    @pl.when(c == cpb - 1)
    def _finish():
        acc = jnp.float32(0.0)
        for r in range(R):
            acc = acc + smax_ref[r] + smin_ref[r]
        o_ref[0, 0, :] = jnp.full((128,), jax.nn.sigmoid(acc),
                                  dtype=jnp.float32)


@jax.jit
def kernel(x, lengths, W):
    B, T, D = x.shape
    nt = T // _TB
    xf = x.reshape(B * T, D)
    pooled = pl.pallas_call(
        _fused_kernel,
        grid=(B * T // _TB,),
        in_specs=[
            pl.BlockSpec(memory_space=pltpu.SMEM),
            pl.BlockSpec((_TB, D), lambda i: (i, 0)),
            pl.BlockSpec((D, 1), lambda i: (0, 0)),
        ],
        out_specs=pl.BlockSpec((1, 1, 128), lambda i: (i // nt, 0, 0)),
        out_shape=jax.ShapeDtypeStruct((B, 1, 128), jnp.float32),
        scratch_shapes=[
            pltpu.SMEM((R,), jnp.float32),
            pltpu.SMEM((R,), jnp.float32),
        ],
    )(lengths, xf, W)
    return pooled[:, 0, 0]
